# baseline (device time: 170364 ns/iter reference)
import os

import jax
import jax.numpy as jnp
from jax import lax
from jax.experimental import pallas as pl
from jax.experimental.pallas import tpu as pltpu

_MODE = os.environ.get("KMODE", "full")
_DIRS = (0,) if _MODE == "commr" else (0, 1)

N_DEV = 4
M = 4096
N = 2048
DHALF = M // 2
RCH = DHALF // N_DEV
K = 2
QR = RCH // K
N_STEPS = 2 * (N_DEV - 1)


def kernel(x):

    def body(x_ref, out_ref, comm_r, comm_l,
             send_r, recv_r, send_l, recv_l):
        my = lax.axis_index("i")
        right = lax.rem(my + 1, N_DEV)
        left = lax.rem(my + N_DEV - 1, N_DEV)

        barrier_sem = pltpu.get_barrier_semaphore()
        for nbr in (left, right):
            pl.semaphore_signal(
                barrier_sem, inc=1,
                device_id=(nbr,), device_id_type=pl.DeviceIdType.MESH,
            )
        pl.semaphore_wait(barrier_sem, 2)

        def ring(d):
            return (
                (comm_r, send_r, recv_r, right) if d == 0
                else (comm_l, send_l, recv_l, left)
            )

        def rows_of(chunk, k, d):
            return pl.ds(d * DHALF + chunk * RCH + k * QR, QR)

        def rs_desc(s, k, chunk, d):
            comm, ssem, rsem, dev = ring(d)
            return pltpu.make_async_remote_copy(
                src_ref=out_ref.at[rows_of(chunk, k, d), :],
                dst_ref=comm.at[s, pl.ds(k * QR, QR), :],
                send_sem=ssem.at[s, k],
                recv_sem=rsem.at[s, k],
                device_id=(dev,),
                device_id_type=pl.DeviceIdType.MESH,
            )

        def ag_desc(t, k, chunk, d):
            comm, ssem, rsem, dev = ring(d)
            sl = (rows_of(chunk, k, d), slice(None))
            return pltpu.make_async_remote_copy(
                src_ref=out_ref.at[sl],
                dst_ref=out_ref.at[sl],
                send_sem=ssem.at[N_DEV - 1 + t, k],
                recv_sem=rsem.at[N_DEV - 1 + t, k],
                device_id=(dev,),
                device_id_type=pl.DeviceIdType.MESH,
            )

        do_comm = _MODE in ("full", "comm", "commr")
        do_comp = _MODE in ("full", "compute")

        if do_comp:
            for d in _DIRS:
                rows = pl.ds(d * DHALF + my * RCH, RCH)
                out_ref[rows, :] = x_ref[0, rows, :].astype(jnp.bfloat16)
        if do_comm:
            for k in range(K):
                for d in _DIRS:
                    rs_desc(0, k, my, d).start()

        for s in range(N_DEV - 1):
            for k in range(K):
                for d in _DIRS:
                    if d == 0:
                        recv_c = lax.rem(my - s - 1 + N_DEV, N_DEV)
                    else:
                        recv_c = lax.rem(my + s + 1, N_DEV)
                    comm = ring(d)[0]
                    if do_comm:
                        rs_desc(s, k, 0, d).wait_recv()
                    if do_comp:
                        rows = rows_of(recv_c, k, d)
                        out_ref[rows, :] = (
                            x_ref[0, rows, :].astype(jnp.bfloat16)
                            + comm[s, pl.ds(k * QR, QR), :]
                        )
                    if do_comm:
                        if s < N_DEV - 2:
                            rs_desc(s + 1, k, recv_c, d).start()
                        else:
                            ag_desc(0, k, recv_c, d).start()

        if do_comm:
            for t in range(N_DEV - 1):
                for k in range(K):
                    for d in _DIRS:
                        if d == 0:
                            recv_c = lax.rem(my - t + N_DEV, N_DEV)
                        else:
                            recv_c = lax.rem(my + t, N_DEV)
                        ag_desc(t, k, recv_c, d).wait_recv()
                        if t < N_DEV - 2:
                            ag_desc(t + 1, k, recv_c, d).start()

            for s in range(N_DEV - 1):
                for k in range(K):
                    for d in _DIRS:
                        rs_desc(s, k, 0, d).wait_send()
            for t in range(N_DEV - 1):
                for k in range(K):
                    for d in _DIRS:
                        ag_desc(t, k, 0, d).wait_send()

    return pl.pallas_call(
        body,
        out_shape=jax.ShapeDtypeStruct((M, N), jnp.bfloat16),
        in_specs=[pl.BlockSpec(memory_space=pltpu.VMEM)],
        out_specs=pl.BlockSpec(memory_space=pltpu.VMEM),
        scratch_shapes=[
            pltpu.VMEM((N_DEV - 1, RCH, N), jnp.bfloat16),
            pltpu.VMEM((N_DEV - 1, RCH, N), jnp.bfloat16),
            pltpu.SemaphoreType.DMA((N_STEPS, K)),
            pltpu.SemaphoreType.DMA((N_STEPS, K)),
            pltpu.SemaphoreType.DMA((N_STEPS, K)),
            pltpu.SemaphoreType.DMA((N_STEPS, K)),
        ],
        compiler_params=pltpu.CompilerParams(
            collective_id=0,
            vmem_limit_bytes=100 * 1024 * 1024,
        ),
    )(x)


# device time: 157437 ns/iter; 1.0821x vs baseline; 1.0821x over previous
import jax
import jax.numpy as jnp
from jax import lax
from jax.experimental import pallas as pl
from jax.experimental.pallas import tpu as pltpu

N_DEV = 4
M = 4096
N = 2048
DHALF = M // 2
RCH = DHALF // N_DEV
K = 2
QR = RCH // K
N_STEPS = 2 * (N_DEV - 1)
PRO = N_DEV - 1


def kernel(x):

    def body(x_hbm, out_hbm, vout, comm_r, comm_l, xstage,
             send_r, recv_r, send_l, recv_l, xsem, fsem):
        my = lax.axis_index("i")
        right = lax.rem(my + 1, N_DEV)
        left = lax.rem(my + N_DEV - 1, N_DEV)

        barrier_sem = pltpu.get_barrier_semaphore()
        for nbr in (left, right):
            pl.semaphore_signal(
                barrier_sem, inc=1,
                device_id=(nbr,), device_id_type=pl.DeviceIdType.MESH,
            )
        pl.semaphore_wait(barrier_sem, 2)

        def ring(d):
            return (
                (comm_r, send_r, recv_r, right) if d == 0
                else (comm_l, send_l, recv_l, left)
            )

        def rows_of(chunk, k, d):
            return pl.ds(d * DHALF + chunk * RCH + k * QR, QR)

        def rs_recv_chunk(s, d):
            if d == 0:
                return lax.rem(my - s - 1 + N_DEV, N_DEV)
            return lax.rem(my + s + 1, N_DEV)

        def ag_recv_chunk(t, d):
            if d == 0:
                return lax.rem(my - t + N_DEV, N_DEV)
            return lax.rem(my + t, N_DEV)

        def x_dma(slot, k, d, chunk):
            return pltpu.make_async_copy(
                x_hbm.at[0, rows_of(chunk, k, d), :],
                xstage.at[slot, d, pl.ds(k * QR, QR), :],
                xsem.at[slot, k, d],
            )

        def flush_dma(slot, k, d, chunk):
            return pltpu.make_async_copy(
                vout.at[rows_of(chunk, k, d), :],
                out_hbm.at[rows_of(chunk, k, d), :],
                fsem.at[slot, k, d],
            )

        def rs_desc(s, k, chunk, d):
            comm, ssem, rsem, dev = ring(d)
            return pltpu.make_async_remote_copy(
                src_ref=vout.at[rows_of(chunk, k, d), :],
                dst_ref=comm.at[s, pl.ds(k * QR, QR), :],
                send_sem=ssem.at[s, k],
                recv_sem=rsem.at[s, k],
                device_id=(dev,),
                device_id_type=pl.DeviceIdType.MESH,
            )

        def ag_desc(t, k, chunk, d):
            comm, ssem, rsem, dev = ring(d)
            sl = (rows_of(chunk, k, d), slice(None))
            return pltpu.make_async_remote_copy(
                src_ref=vout.at[sl],
                dst_ref=vout.at[sl],
                send_sem=ssem.at[N_DEV - 1 + t, k],
                recv_sem=rsem.at[N_DEV - 1 + t, k],
                device_id=(dev,),
                device_id_type=pl.DeviceIdType.MESH,
            )

        for k in range(K):
            for d in (0, 1):
                x_dma(PRO, k, d, my).start()
        for s in range(N_DEV - 1):
            for k in range(K):
                for d in (0, 1):
                    x_dma(s, k, d, rs_recv_chunk(s, d)).start()

        for k in range(K):
            for d in (0, 1):
                x_dma(PRO, k, d, my).wait()
                rows = rows_of(my, k, d)
                vout[rows, :] = (
                    xstage[PRO, d, pl.ds(k * QR, QR), :].astype(jnp.bfloat16)
                )
                rs_desc(0, k, my, d).start()

        for s in range(N_DEV - 1):
            for k in range(K):
                for d in (0, 1):
                    recv_c = rs_recv_chunk(s, d)
                    comm = ring(d)[0]
                    rs_desc(s, k, 0, d).wait_recv()
                    x_dma(s, k, d, recv_c).wait()
                    rows = rows_of(recv_c, k, d)
                    vout[rows, :] = (
                        xstage[s, d, pl.ds(k * QR, QR), :].astype(jnp.bfloat16)
                        + comm[s, pl.ds(k * QR, QR), :]
                    )
                    if s < N_DEV - 2:
                        rs_desc(s + 1, k, recv_c, d).start()
                    else:
                        ag_desc(0, k, recv_c, d).start()
                        flush_dma(PRO, k, d, recv_c).start()

        for t in range(N_DEV - 1):
            for k in range(K):
                for d in (0, 1):
                    recv_c = ag_recv_chunk(t, d)
                    ag_desc(t, k, recv_c, d).wait_recv()
                    if t < N_DEV - 2:
                        ag_desc(t + 1, k, recv_c, d).start()
                    flush_dma(t, k, d, recv_c).start()

        for slot in range(N_DEV):
            for k in range(K):
                for d in (0, 1):
                    chunk = (
                        ag_recv_chunk(slot, d) if slot < N_DEV - 1
                        else rs_recv_chunk(N_DEV - 2, d)
                    )
                    flush_dma(slot, k, d, chunk).wait()
        for s in range(N_DEV - 1):
            for k in range(K):
                for d in (0, 1):
                    rs_desc(s, k, 0, d).wait_send()
        for t in range(N_DEV - 1):
            for k in range(K):
                for d in (0, 1):
                    ag_desc(t, k, 0, d).wait_send()

    return pl.pallas_call(
        body,
        out_shape=jax.ShapeDtypeStruct((M, N), jnp.bfloat16),
        in_specs=[pl.BlockSpec(memory_space=pl.ANY)],
        out_specs=pl.BlockSpec(memory_space=pl.ANY),
        scratch_shapes=[
            pltpu.VMEM((M, N), jnp.bfloat16),
            pltpu.VMEM((N_DEV - 1, RCH, N), jnp.bfloat16),
            pltpu.VMEM((N_DEV - 1, RCH, N), jnp.bfloat16),
            pltpu.VMEM((N_DEV, 2, RCH, N), jnp.float32),
            pltpu.SemaphoreType.DMA((N_STEPS, K)),
            pltpu.SemaphoreType.DMA((N_STEPS, K)),
            pltpu.SemaphoreType.DMA((N_STEPS, K)),
            pltpu.SemaphoreType.DMA((N_STEPS, K)),
            pltpu.SemaphoreType.DMA((N_DEV, K, 2)),
            pltpu.SemaphoreType.DMA((N_DEV, K, 2)),
        ],
        compiler_params=pltpu.CompilerParams(
            collective_id=0,
            vmem_limit_bytes=100 * 1024 * 1024,
        ),
    )(x)


# device time: 156084 ns/iter; 1.0915x vs baseline; 1.0087x over previous
import jax
import jax.numpy as jnp
from jax import lax
from jax.experimental import pallas as pl
from jax.experimental.pallas import tpu as pltpu

N_DEV = 4
M = 4096
N = 2048
DHALF = M // 2
RCH = DHALF // N_DEV
K = 4
QR = RCH // K
N_STEPS = 2 * (N_DEV - 1)
PRO = N_DEV - 1


def kernel(x):

    def body(x_hbm, out_hbm, vout, comm_r, comm_l, xstage,
             send_r, recv_r, send_l, recv_l, xsem, fsem):
        my = lax.axis_index("i")
        right = lax.rem(my + 1, N_DEV)
        left = lax.rem(my + N_DEV - 1, N_DEV)

        def ring(d):
            return (
                (comm_r, send_r, recv_r, right) if d == 0
                else (comm_l, send_l, recv_l, left)
            )

        def rows_of(chunk, k, d):
            return pl.ds(d * DHALF + chunk * RCH + k * QR, QR)

        def rs_recv_chunk(s, d):
            if d == 0:
                return lax.rem(my - s - 1 + N_DEV, N_DEV)
            return lax.rem(my + s + 1, N_DEV)

        def ag_recv_chunk(t, d):
            if d == 0:
                return lax.rem(my - t + N_DEV, N_DEV)
            return lax.rem(my + t, N_DEV)

        def x_dma(slot, k, d, chunk):
            return pltpu.make_async_copy(
                x_hbm.at[0, rows_of(chunk, k, d), :],
                xstage.at[slot, d, pl.ds(k * QR, QR), :],
                xsem.at[slot, k, d],
            )

        def flush_dma(slot, k, d, chunk):
            return pltpu.make_async_copy(
                vout.at[rows_of(chunk, k, d), :],
                out_hbm.at[rows_of(chunk, k, d), :],
                fsem.at[slot, k, d],
            )

        def rs_desc(s, k, chunk, d):
            comm, ssem, rsem, dev = ring(d)
            return pltpu.make_async_remote_copy(
                src_ref=vout.at[rows_of(chunk, k, d), :],
                dst_ref=comm.at[s, pl.ds(k * QR, QR), :],
                send_sem=ssem.at[s, k],
                recv_sem=rsem.at[s, k],
                device_id=(dev,),
                device_id_type=pl.DeviceIdType.MESH,
            )

        def ag_desc(t, k, chunk, d):
            comm, ssem, rsem, dev = ring(d)
            sl = (rows_of(chunk, k, d), slice(None))
            return pltpu.make_async_remote_copy(
                src_ref=vout.at[sl],
                dst_ref=vout.at[sl],
                send_sem=ssem.at[N_DEV - 1 + t, k],
                recv_sem=rsem.at[N_DEV - 1 + t, k],
                device_id=(dev,),
                device_id_type=pl.DeviceIdType.MESH,
            )

        for k in range(K):
            for d in (0, 1):
                x_dma(PRO, k, d, my).start()
        for s in range(N_DEV - 1):
            for k in range(K):
                for d in (0, 1):
                    x_dma(s, k, d, rs_recv_chunk(s, d)).start()

        barrier_sem = pltpu.get_barrier_semaphore()
        for nbr in (left, right):
            pl.semaphore_signal(
                barrier_sem, inc=1,
                device_id=(nbr,), device_id_type=pl.DeviceIdType.MESH,
            )
        pl.semaphore_wait(barrier_sem, 2)

        for k in range(K):
            for d in (0, 1):
                x_dma(PRO, k, d, my).wait()
                rows = rows_of(my, k, d)
                vout[rows, :] = (
                    xstage[PRO, d, pl.ds(k * QR, QR), :].astype(jnp.bfloat16)
                )
                rs_desc(0, k, my, d).start()

        for s in range(N_DEV - 1):
            for k in range(K):
                for d in (0, 1):
                    recv_c = rs_recv_chunk(s, d)
                    comm = ring(d)[0]
                    rs_desc(s, k, 0, d).wait_recv()
                    x_dma(s, k, d, recv_c).wait()
                    rows = rows_of(recv_c, k, d)
                    vout[rows, :] = (
                        xstage[s, d, pl.ds(k * QR, QR), :].astype(jnp.bfloat16)
                        + comm[s, pl.ds(k * QR, QR), :]
                    )
                    if s < N_DEV - 2:
                        rs_desc(s + 1, k, recv_c, d).start()
                    else:
                        ag_desc(0, k, recv_c, d).start()
                        flush_dma(PRO, k, d, recv_c).start()

        for t in range(N_DEV - 1):
            for k in range(K):
                for d in (0, 1):
                    recv_c = ag_recv_chunk(t, d)
                    ag_desc(t, k, recv_c, d).wait_recv()
                    if t < N_DEV - 2:
                        ag_desc(t + 1, k, recv_c, d).start()
                    flush_dma(t, k, d, recv_c).start()

        for slot in range(N_DEV):
            for k in range(K):
                for d in (0, 1):
                    chunk = (
                        ag_recv_chunk(slot, d) if slot < N_DEV - 1
                        else rs_recv_chunk(N_DEV - 2, d)
                    )
                    flush_dma(slot, k, d, chunk).wait()
        for s in range(N_DEV - 1):
            for k in range(K):
                for d in (0, 1):
                    rs_desc(s, k, 0, d).wait_send()
        for t in range(N_DEV - 1):
            for k in range(K):
                for d in (0, 1):
                    ag_desc(t, k, 0, d).wait_send()

    return pl.pallas_call(
        body,
        out_shape=jax.ShapeDtypeStruct((M, N), jnp.bfloat16),
        in_specs=[pl.BlockSpec(memory_space=pl.ANY)],
        out_specs=pl.BlockSpec(memory_space=pl.ANY),
        scratch_shapes=[
            pltpu.VMEM((M, N), jnp.bfloat16),
            pltpu.VMEM((N_DEV - 1, RCH, N), jnp.bfloat16),
            pltpu.VMEM((N_DEV - 1, RCH, N), jnp.bfloat16),
            pltpu.VMEM((N_DEV, 2, RCH, N), jnp.float32),
            pltpu.SemaphoreType.DMA((N_STEPS, K)),
            pltpu.SemaphoreType.DMA((N_STEPS, K)),
            pltpu.SemaphoreType.DMA((N_STEPS, K)),
            pltpu.SemaphoreType.DMA((N_STEPS, K)),
            pltpu.SemaphoreType.DMA((N_DEV, K, 2)),
            pltpu.SemaphoreType.DMA((N_DEV, K, 2)),
        ],
        compiler_params=pltpu.CompilerParams(
            collective_id=0,
            vmem_limit_bytes=100 * 1024 * 1024,
        ),
    )(x)
